# baseline (speedup 1.0000x reference)
import functools

import jax
import jax.numpy as jnp
from jax import lax
from jax.experimental import pallas as pl
from jax.experimental.pallas import tpu as pltpu
from jax.experimental.pallas import tpu_sc as plsc

E = 8
KTOP = 2
D = 768
I = 1536
T = 2048

NC, NS, L = 2, 16, 16
NW = NC * NS
CHUNK = T // NW
NV = T // L

BM = 128
NT = (KTOP * T) // BM + E
MR = NT * BM
NTE = NT


def _route_body(lt_ref, i0_ref, i1_ref, w0_ref, w1_ref, cbs_ref, te_ref):
    l = lt_ref[...]
    iota = lax.broadcasted_iota(jnp.int32, (E, T), 0)
    m1 = jnp.max(l, axis=0, keepdims=True)
    top1 = jnp.min(jnp.where(l >= m1, iota, E), axis=0, keepdims=True)
    lm = jnp.where(iota == top1, -jnp.inf, l)
    m2 = jnp.max(lm, axis=0, keepdims=True)
    top2 = jnp.min(jnp.where(lm >= m2, iota, E), axis=0, keepdims=True)
    t = jnp.exp(m2 - m1)
    w0 = 1.0 / (1.0 + t)
    i0_ref[...] = top1.reshape(T)
    i1_ref[...] = top2.reshape(T)
    w0_ref[...] = w0.reshape(T)
    w1_ref[...] = (1.0 - w0).reshape(T)

    bmat = ((iota == top1).astype(jnp.float32)
            + (iota == top2).astype(jnp.float32))
    s2 = (lax.broadcasted_iota(jnp.int32, (NW, T), 0)
          == lax.broadcasted_iota(jnp.int32, (NW, T), 1) // CHUNK
          ).astype(jnp.float32)
    hist_t = lax.dot_general(s2, bmat, (((1,), (1,)), ((), ())),
                             preferred_element_type=jnp.float32)
    ltri = (lax.broadcasted_iota(jnp.int32, (NW, NW), 1)
            < lax.broadcasted_iota(jnp.int32, (NW, NW), 0)
            ).astype(jnp.float32)
    pc_t = lax.dot_general(ltri, hist_t, (((1,), (0,)), ((), ())),
                           preferred_element_type=jnp.float32)
    tot1 = jnp.sum(hist_t, axis=0, keepdims=True)
    g1 = (((tot1.astype(jnp.int32) + (BM - 1)) // BM) * BM).astype(jnp.float32)
    u8 = (lax.broadcasted_iota(jnp.int32, (E, E), 0)
          < lax.broadcasted_iota(jnp.int32, (E, E), 1)).astype(jnp.float32)
    base1 = lax.dot_general(g1, u8, (((1,), (0,)), ((), ())),
                            preferred_element_type=jnp.float32)
    cb_t = (pc_t + base1).astype(jnp.int32)
    cbs_ref[...] = jnp.concatenate(
        [cb_t, jnp.zeros((NW, 3 * E), jnp.int32)], axis=1)

    hist8 = lax.dot_general(bmat, s2, (((1,), (1,)), ((), ())),
                            preferred_element_type=jnp.float32)
    tot8 = jnp.sum(hist8, axis=1, keepdims=True)
    g8 = (((tot8.astype(jnp.int32) + (BM - 1)) // BM) * BM).astype(jnp.float32)
    l8 = (lax.broadcasted_iota(jnp.int32, (E, E), 1)
          < lax.broadcasted_iota(jnp.int32, (E, E), 0)).astype(jnp.float32)
    base8 = lax.dot_general(l8, g8, (((1,), (0,)), ((), ())),
                            preferred_element_type=jnp.float32)
    s_bm = (lax.broadcasted_iota(jnp.int32, (1, NTE), 1) * BM).astype(jnp.float32)
    cnt = jnp.sum((base8 <= s_bm).astype(jnp.float32), axis=0, keepdims=True)
    te = jnp.minimum(cnt.astype(jnp.int32) - 1, E - 1)
    te_ref[...] = te.reshape(NTE)


def _route(logits_t):
    return pl.pallas_call(
        _route_body,
        out_shape=(
            jax.ShapeDtypeStruct((T,), jnp.int32),
            jax.ShapeDtypeStruct((T,), jnp.int32),
            jax.ShapeDtypeStruct((T,), jnp.float32),
            jax.ShapeDtypeStruct((T,), jnp.float32),
            jax.ShapeDtypeStruct((NW, 4 * E), jnp.int32),
            jax.ShapeDtypeStruct((NTE,), jnp.int32),
        ),
    )(logits_t)


def _dispatch_body(i0_hbm, i1_hbm, w0_hbm, w1_hbm, hid_hbm, cbs_hbm,
                   xs_hbm, ws_hbm, p0_hbm, p1_hbm,
                   c0_v, c1_v, xbuf_v, wbuf0_v, wbuf1_v, p0_v, p1_v, cb_v,
                   cb_sm, sem0, sem1, sem2, sem3, sem4, sem5):
    wid = lax.axis_index("s") * NC + lax.axis_index("c")
    tbase = wid * CHUNK
    ds = pl.ds(tbase, CHUNK)

    cp0 = pltpu.async_copy(i0_hbm.at[ds], c0_v, sem0)
    cp1 = pltpu.async_copy(i1_hbm.at[ds], c1_v, sem1)
    cpc = pltpu.async_copy(cbs_hbm.at[wid], cb_v, sem2)
    cpx = pltpu.async_copy(hid_hbm.at[ds], xbuf_v, sem3)
    cpw0 = pltpu.async_copy(w0_hbm.at[ds], wbuf0_v, sem4)
    cpw1 = pltpu.async_copy(w1_hbm.at[ds], wbuf1_v, sem5)
    cp0.wait()
    cp1.wait()
    cpc.wait()
    cbv = cb_v[pl.ds(0, L)]
    for e in range(E):
        cb_sm[e] = cbv[e]

    def place(ids_ref, pos_ref):
        for g in range(CHUNK // L):
            v = ids_ref[pl.ds(g * L, L)]
            posv = jnp.zeros((L,), jnp.int32)
            for e in range(E):
                m = (v == jnp.full((L,), e, jnp.int32)).astype(jnp.int32)
                rank = plsc.cumsum(m) - m
                c = cb_sm[e]
                posv = posv + m * (jnp.full((L,), c, jnp.int32) + rank)
                cb_sm[e] = c + jnp.sum(m)
            pos_ref[pl.ds(g * L, L)] = posv

    place(c0_v, p0_v)
    place(c1_v, p1_v)

    o0 = pltpu.async_copy(p0_v, p0_hbm.at[ds], sem0)
    o1 = pltpu.async_copy(p1_v, p1_hbm.at[ds], sem1)
    cpx.wait()
    o2 = pltpu.async_copy(xbuf_v, xs_hbm.at[p0_v], sem2)
    o3 = pltpu.async_copy(xbuf_v, xs_hbm.at[p1_v], sem3)
    cpw0.wait()
    cpw1.wait()
    o4 = pltpu.async_copy(wbuf0_v, ws_hbm.at[p0_v], sem4)
    o5 = pltpu.async_copy(wbuf1_v, ws_hbm.at[p1_v], sem5)
    o0.wait()
    o1.wait()
    o2.wait()
    o3.wait()
    o4.wait()
    o5.wait()


def _dispatch(ids0, ids1, w0, w1, hidden, cbs):
    mesh = plsc.VectorSubcoreMesh(core_axis_name="c", subcore_axis_name="s",
                                  num_cores=NC, num_subcores=NS)
    return pl.kernel(
        _dispatch_body,
        out_type=(
            jax.ShapeDtypeStruct((MR, D), jnp.float32),
            jax.ShapeDtypeStruct((MR,), jnp.float32),
            jax.ShapeDtypeStruct((T,), jnp.int32),
            jax.ShapeDtypeStruct((T,), jnp.int32),
        ),
        mesh=mesh,
        compiler_params=pltpu.CompilerParams(needs_layout_passes=False),
        scratch_types=(
            pltpu.VMEM((CHUNK,), jnp.int32),
            pltpu.VMEM((CHUNK,), jnp.int32),
            pltpu.VMEM((CHUNK, D), jnp.float32),
            pltpu.VMEM((CHUNK,), jnp.float32),
            pltpu.VMEM((CHUNK,), jnp.float32),
            pltpu.VMEM((CHUNK,), jnp.int32),
            pltpu.VMEM((CHUNK,), jnp.int32),
            pltpu.VMEM((4 * E,), jnp.int32),
            pltpu.SMEM((E,), jnp.int32),
            pltpu.SemaphoreType.DMA,
            pltpu.SemaphoreType.DMA,
            pltpu.SemaphoreType.DMA,
            pltpu.SemaphoreType.DMA,
            pltpu.SemaphoreType.DMA,
            pltpu.SemaphoreType.DMA,
        ),
    )(ids0, ids1, w0, w1, hidden, cbs)


def _mlp_body(te_ref, x_ref, w13_ref, w2_ref, ws_ref, y_ref):
    x = x_ref[...]
    dn = (((1,), (1,)), ((), ()))
    g = lax.dot_general(x, w13_ref[0, :I, :], dn,
                        preferred_element_type=jnp.float32)
    u = lax.dot_general(x, w13_ref[0, I:, :], dn,
                        preferred_element_type=jnp.float32)
    h = g * u / (1.0 + jnp.exp(-g))
    y = lax.dot_general(h, w2_ref[0], dn, preferred_element_type=jnp.float32)
    y_ref[...] = y * ws_ref[...].reshape(BM, 1)


def _mlp(te, xs, w13, w2, ws):
    grid_spec = pltpu.PrefetchScalarGridSpec(
        num_scalar_prefetch=1,
        grid=(NT,),
        in_specs=[
            pl.BlockSpec((BM, D), lambda i, te: (i, 0)),
            pl.BlockSpec((1, 2 * I, D), lambda i, te: (te[i], 0, 0)),
            pl.BlockSpec((1, D, I), lambda i, te: (te[i], 0, 0)),
            pl.BlockSpec((BM,), lambda i, te: (i,)),
        ],
        out_specs=pl.BlockSpec((BM, D), lambda i, te: (i, 0)),
    )
    return pl.pallas_call(
        _mlp_body,
        grid_spec=grid_spec,
        out_shape=jax.ShapeDtypeStruct((MR, D), jnp.float32),
    )(te, xs, w13, w2, ws)


def _combine_body(y_hbm, p0_hbm, p1_hbm, out_hbm, p0_v, p1_v, r0_v, r1_v,
                  sem0, sem1):
    wid = lax.axis_index("s") * NC + lax.axis_index("c")
    tbase = wid * CHUNK
    pltpu.sync_copy(p0_hbm.at[pl.ds(tbase, CHUNK)], p0_v)
    pltpu.sync_copy(p1_hbm.at[pl.ds(tbase, CHUNK)], p1_v)
    c0 = pltpu.async_copy(y_hbm.at[p0_v], r0_v, sem0)
    c1 = pltpu.async_copy(y_hbm.at[p1_v], r1_v, sem1)
    c0.wait()
    c1.wait()

    def add_row(i, _):
        for c in range(D // L):
            s = pl.ds(c * L, L)
            r0_v[i, s] = r0_v[i, s] + r1_v[i, s]
        return 0
    lax.fori_loop(0, CHUNK, add_row, 0)
    pltpu.sync_copy(r0_v, out_hbm.at[pl.ds(tbase, CHUNK)])


def _combine(y, p0, p1):
    mesh = plsc.VectorSubcoreMesh(core_axis_name="c", subcore_axis_name="s",
                                  num_cores=NC, num_subcores=NS)
    return pl.kernel(
        _combine_body,
        out_type=jax.ShapeDtypeStruct((T, D), jnp.float32),
        mesh=mesh,
        compiler_params=pltpu.CompilerParams(needs_layout_passes=False),
        scratch_types=(
            pltpu.VMEM((CHUNK,), jnp.int32),
            pltpu.VMEM((CHUNK,), jnp.int32),
            pltpu.VMEM((CHUNK, D), jnp.float32),
            pltpu.VMEM((CHUNK, D), jnp.float32),
            pltpu.SemaphoreType.DMA,
            pltpu.SemaphoreType.DMA,
        ),
    )(y, p0, p1)


def kernel(hidden_states, router_logits, w13_weight, w2_weight):
    ids0, ids1, w0, w1, cbs, te = _route(router_logits.T)
    xs, ws, p0, p1 = _dispatch(ids0, ids1, w0, w1, hidden_states, cbs)
    y = _mlp(te, xs, w13_weight, w2_weight, ws)
    return _combine(y, p0, p1)


# speedup vs baseline: 1.3127x; 1.3127x over previous
import functools

import jax
import jax.numpy as jnp
from jax import lax
from jax.experimental import pallas as pl
from jax.experimental.pallas import tpu as pltpu
from jax.experimental.pallas import tpu_sc as plsc

E = 8
KTOP = 2
D = 768
I = 1536
T = 2048

NC, NS, L = 2, 16, 16
NW = NC * NS
CHUNK = T // NW
NV = T // L

BM = 256
NT = (KTOP * T) // BM + E
MR = NT * BM
NTE = NT


def _route_body(lt_ref, i0_ref, i1_ref, w0_ref, w1_ref, cbs_ref, te_ref):
    l = lt_ref[...]
    iota = lax.broadcasted_iota(jnp.int32, (E, T), 0)
    m1 = jnp.max(l, axis=0, keepdims=True)
    top1 = jnp.min(jnp.where(l >= m1, iota, E), axis=0, keepdims=True)
    lm = jnp.where(iota == top1, -jnp.inf, l)
    m2 = jnp.max(lm, axis=0, keepdims=True)
    top2 = jnp.min(jnp.where(lm >= m2, iota, E), axis=0, keepdims=True)
    t = jnp.exp(m2 - m1)
    w0 = 1.0 / (1.0 + t)
    i0_ref[...] = top1.reshape(T)
    i1_ref[...] = top2.reshape(T)
    w0_ref[...] = w0.reshape(T)
    w1_ref[...] = (1.0 - w0).reshape(T)

    bmat = ((iota == top1).astype(jnp.float32)
            + (iota == top2).astype(jnp.float32))
    s2 = (lax.broadcasted_iota(jnp.int32, (NW, T), 0)
          == lax.broadcasted_iota(jnp.int32, (NW, T), 1) // CHUNK
          ).astype(jnp.float32)
    hist_t = lax.dot_general(s2, bmat, (((1,), (1,)), ((), ())),
                             preferred_element_type=jnp.float32)
    ltri = (lax.broadcasted_iota(jnp.int32, (NW, NW), 1)
            < lax.broadcasted_iota(jnp.int32, (NW, NW), 0)
            ).astype(jnp.float32)
    pc_t = lax.dot_general(ltri, hist_t, (((1,), (0,)), ((), ())),
                           preferred_element_type=jnp.float32)
    tot1 = jnp.sum(hist_t, axis=0, keepdims=True)
    g1 = (((tot1.astype(jnp.int32) + (BM - 1)) // BM) * BM).astype(jnp.float32)
    u8 = (lax.broadcasted_iota(jnp.int32, (E, E), 0)
          < lax.broadcasted_iota(jnp.int32, (E, E), 1)).astype(jnp.float32)
    base1 = lax.dot_general(g1, u8, (((1,), (0,)), ((), ())),
                            preferred_element_type=jnp.float32)
    cb_t = (pc_t + base1).astype(jnp.int32)
    cbs_ref[...] = jnp.concatenate(
        [cb_t, jnp.zeros((NW, 3 * E), jnp.int32)], axis=1)

    hist8 = lax.dot_general(bmat, s2, (((1,), (1,)), ((), ())),
                            preferred_element_type=jnp.float32)
    tot8 = jnp.sum(hist8, axis=1, keepdims=True)
    g8 = (((tot8.astype(jnp.int32) + (BM - 1)) // BM) * BM).astype(jnp.float32)
    l8 = (lax.broadcasted_iota(jnp.int32, (E, E), 1)
          < lax.broadcasted_iota(jnp.int32, (E, E), 0)).astype(jnp.float32)
    base8 = lax.dot_general(l8, g8, (((1,), (0,)), ((), ())),
                            preferred_element_type=jnp.float32)
    s_bm = (lax.broadcasted_iota(jnp.int32, (1, NTE), 1) * BM).astype(jnp.float32)
    cnt = jnp.sum((base8 <= s_bm).astype(jnp.float32), axis=0, keepdims=True)
    te = jnp.minimum(cnt.astype(jnp.int32) - 1, E - 1)
    te_ref[...] = te.reshape(NTE)


def _route(logits_t):
    return pl.pallas_call(
        _route_body,
        out_shape=(
            jax.ShapeDtypeStruct((T,), jnp.int32),
            jax.ShapeDtypeStruct((T,), jnp.int32),
            jax.ShapeDtypeStruct((T,), jnp.float32),
            jax.ShapeDtypeStruct((T,), jnp.float32),
            jax.ShapeDtypeStruct((NW, 4 * E), jnp.int32),
            jax.ShapeDtypeStruct((NTE,), jnp.int32),
        ),
    )(logits_t)


def _dispatch_body(i0_hbm, i1_hbm, w0_hbm, w1_hbm, hid_hbm, cbs_hbm,
                   xs_hbm, ws_hbm, p0_hbm, p1_hbm,
                   c0_v, c1_v, xbuf_v, wbuf0_v, wbuf1_v, p0_v, p1_v, cb_v,
                   cb_sm, sem0, sem1, sem2, sem3, sem4, sem5):
    wid = lax.axis_index("s") * NC + lax.axis_index("c")
    tbase = wid * CHUNK
    ds = pl.ds(tbase, CHUNK)

    cp0 = pltpu.async_copy(i0_hbm.at[ds], c0_v, sem0)
    cp1 = pltpu.async_copy(i1_hbm.at[ds], c1_v, sem1)
    cpc = pltpu.async_copy(cbs_hbm.at[wid], cb_v, sem2)
    cpx = pltpu.async_copy(hid_hbm.at[ds], xbuf_v, sem3)
    cpw0 = pltpu.async_copy(w0_hbm.at[ds], wbuf0_v, sem4)
    cpw1 = pltpu.async_copy(w1_hbm.at[ds], wbuf1_v, sem5)
    cp0.wait()
    cp1.wait()
    cpc.wait()
    cbv = cb_v[pl.ds(0, L)]
    for e in range(E):
        cb_sm[e] = cbv[e]

    def place(ids_ref, pos_ref):
        for g in range(CHUNK // L):
            v = ids_ref[pl.ds(g * L, L)]
            posv = jnp.zeros((L,), jnp.int32)
            for e in range(E):
                m = (v == jnp.full((L,), e, jnp.int32)).astype(jnp.int32)
                rank = plsc.cumsum(m) - m
                c = cb_sm[e]
                posv = posv + m * (jnp.full((L,), c, jnp.int32) + rank)
                cb_sm[e] = c + jnp.sum(m)
            pos_ref[pl.ds(g * L, L)] = posv

    place(c0_v, p0_v)
    place(c1_v, p1_v)

    o0 = pltpu.async_copy(p0_v, p0_hbm.at[ds], sem0)
    o1 = pltpu.async_copy(p1_v, p1_hbm.at[ds], sem1)
    cpx.wait()
    o2 = pltpu.async_copy(xbuf_v, xs_hbm.at[p0_v], sem2)
    o3 = pltpu.async_copy(xbuf_v, xs_hbm.at[p1_v], sem3)
    cpw0.wait()
    cpw1.wait()
    o4 = pltpu.async_copy(wbuf0_v, ws_hbm.at[p0_v], sem4)
    o5 = pltpu.async_copy(wbuf1_v, ws_hbm.at[p1_v], sem5)
    o0.wait()
    o1.wait()
    o2.wait()
    o3.wait()
    o4.wait()
    o5.wait()


def _dispatch(ids0, ids1, w0, w1, hidden, cbs):
    mesh = plsc.VectorSubcoreMesh(core_axis_name="c", subcore_axis_name="s",
                                  num_cores=NC, num_subcores=NS)
    return pl.kernel(
        _dispatch_body,
        out_type=(
            jax.ShapeDtypeStruct((MR, D), jnp.float32),
            jax.ShapeDtypeStruct((MR,), jnp.float32),
            jax.ShapeDtypeStruct((T,), jnp.int32),
            jax.ShapeDtypeStruct((T,), jnp.int32),
        ),
        mesh=mesh,
        compiler_params=pltpu.CompilerParams(needs_layout_passes=False),
        scratch_types=(
            pltpu.VMEM((CHUNK,), jnp.int32),
            pltpu.VMEM((CHUNK,), jnp.int32),
            pltpu.VMEM((CHUNK, D), jnp.float32),
            pltpu.VMEM((CHUNK,), jnp.float32),
            pltpu.VMEM((CHUNK,), jnp.float32),
            pltpu.VMEM((CHUNK,), jnp.int32),
            pltpu.VMEM((CHUNK,), jnp.int32),
            pltpu.VMEM((4 * E,), jnp.int32),
            pltpu.SMEM((E,), jnp.int32),
            pltpu.SemaphoreType.DMA,
            pltpu.SemaphoreType.DMA,
            pltpu.SemaphoreType.DMA,
            pltpu.SemaphoreType.DMA,
            pltpu.SemaphoreType.DMA,
            pltpu.SemaphoreType.DMA,
        ),
    )(ids0, ids1, w0, w1, hidden, cbs)


def _mlp_body(te_ref, x_ref, w13_ref, w2_ref, ws_ref, y_ref):
    x = x_ref[...]
    dn = (((1,), (1,)), ((), ()))
    g = lax.dot_general(x, w13_ref[0, :I, :], dn,
                        preferred_element_type=jnp.float32)
    u = lax.dot_general(x, w13_ref[0, I:, :], dn,
                        preferred_element_type=jnp.float32)
    h = g * u / (1.0 + jnp.exp(-g))
    y = lax.dot_general(h, w2_ref[0], dn, preferred_element_type=jnp.float32)
    y_ref[...] = y * ws_ref[...].reshape(BM, 1)


def _mlp(te, xs, w13, w2, ws):
    grid_spec = pltpu.PrefetchScalarGridSpec(
        num_scalar_prefetch=1,
        grid=(NT,),
        in_specs=[
            pl.BlockSpec((BM, D), lambda i, te: (i, 0)),
            pl.BlockSpec((1, 2 * I, D), lambda i, te: (te[i], 0, 0)),
            pl.BlockSpec((1, D, I), lambda i, te: (te[i], 0, 0)),
            pl.BlockSpec((BM,), lambda i, te: (i,)),
        ],
        out_specs=pl.BlockSpec((BM, D), lambda i, te: (i, 0)),
    )
    return pl.pallas_call(
        _mlp_body,
        grid_spec=grid_spec,
        out_shape=jax.ShapeDtypeStruct((MR, D), jnp.float32),
    )(te, xs, w13, w2, ws)


def _combine_body(y_hbm, p0_hbm, p1_hbm, out_hbm, p0_v, p1_v, r0_v, r1_v,
                  sem0, sem1):
    wid = lax.axis_index("s") * NC + lax.axis_index("c")
    tbase = wid * CHUNK
    pltpu.sync_copy(p0_hbm.at[pl.ds(tbase, CHUNK)], p0_v)
    pltpu.sync_copy(p1_hbm.at[pl.ds(tbase, CHUNK)], p1_v)
    c0 = pltpu.async_copy(y_hbm.at[p0_v], r0_v, sem0)
    c1 = pltpu.async_copy(y_hbm.at[p1_v], r1_v, sem1)
    c0.wait()
    c1.wait()

    def add_row(i, _):
        for c in range(D // L):
            s = pl.ds(c * L, L)
            r0_v[i, s] = r0_v[i, s] + r1_v[i, s]
        return 0
    lax.fori_loop(0, CHUNK, add_row, 0)
    pltpu.sync_copy(r0_v, out_hbm.at[pl.ds(tbase, CHUNK)])


def _combine(y, p0, p1):
    mesh = plsc.VectorSubcoreMesh(core_axis_name="c", subcore_axis_name="s",
                                  num_cores=NC, num_subcores=NS)
    return pl.kernel(
        _combine_body,
        out_type=jax.ShapeDtypeStruct((T, D), jnp.float32),
        mesh=mesh,
        compiler_params=pltpu.CompilerParams(needs_layout_passes=False),
        scratch_types=(
            pltpu.VMEM((CHUNK,), jnp.int32),
            pltpu.VMEM((CHUNK,), jnp.int32),
            pltpu.VMEM((CHUNK, D), jnp.float32),
            pltpu.VMEM((CHUNK, D), jnp.float32),
            pltpu.SemaphoreType.DMA,
            pltpu.SemaphoreType.DMA,
        ),
    )(y, p0, p1)


def kernel(hidden_states, router_logits, w13_weight, w2_weight):
    ids0, ids1, w0, w1, cbs, te = _route(router_logits.T)
    xs, ws, p0, p1 = _dispatch(ids0, ids1, w0, w1, hidden_states, cbs)
    y = _mlp(te, xs, w13_weight, w2_weight, ws)
    return _combine(y, p0, p1)
